# Initial kernel scaffold; baseline (speedup 1.0000x reference)
import jax
import jax.numpy as jnp
from jax.experimental import pallas as pl


def kernel(uv, feat_2d, feat_3d, interp_out_W, interp_out_b, score1_W, score1_b, score2_W, score2_b, mlps3d_W, mlps3d_b, f2_a1_W, f2_a1_b, f2_a2_W, f2_a2_b, f2_mid_W, f2_out_W, f3_a1_W, f3_a1_b, f3_a2_W, f3_a2_b, f3_mid_W, f3_out_W):
    raise NotImplementedError("write your pallas kernel here")


# speedup vs baseline: 38.1829x; 38.1829x over previous
import jax
import jax.numpy as jnp
from jax.experimental import pallas as pl
from jax.experimental.pallas import tpu as pltpu

_BS, _C2D, _C3D, _H, _W, _N = 2, 64, 64, 64, 64, 4096
_HW = _H * _W
_P = 512
_NC = _HW // _P


def _lrelu(x):
    return jnp.where(x >= 0, x, 0.1 * x)


def _pair_softmax(v0, v1):
    m = jnp.maximum(v0, v1)
    e0 = jnp.exp(v0 - m)
    e1 = jnp.exp(v1 - m)
    inv = 1.0 / (e0 + e1)
    return e0 * inv, e1 * inv


def _interp_kernel(s1_ref, uv_ref, cat_ref, s2w_ref, s2b_ref, iw_ref, ib_ref,
                   out_ref):
    c = pl.program_id(1)
    base = c * _P
    ux = uv_ref[0, 0:1, :]
    uy = uv_ref[0, 1:2, :]
    pix = base + jax.lax.broadcasted_iota(jnp.int32, (_P, 1), 0)
    gx = (pix % _W).astype(jnp.float32)
    gy = (pix // _W).astype(jnp.float32)
    dx = gx - ux
    dy = gy - uy
    dist = dx * dx + dy * dy
    dmin = jnp.min(dist, axis=1, keepdims=True)
    ion = jax.lax.broadcasted_iota(jnp.int32, (_P, _N), 1)
    idx = jnp.min(jnp.where(dist == dmin, ion, _N), axis=1, keepdims=True)
    oh = (ion == idx).astype(jnp.float32)
    g = jnp.dot(oh, cat_ref[0], preferred_element_type=jnp.float32)
    offx = g[:, 0:1] - gx
    offy = g[:, 1:2] - gy
    knn_f3 = g[:, 2:2 + _C3D]
    nrm = jnp.sqrt(offx * offx + offy * offy)
    s = _lrelu(offx * s1_ref[0, 0] + offy * s1_ref[0, 1]
               + nrm * s1_ref[0, 2] + s1_ref[0, 3])
    score = jax.nn.sigmoid(s * s2w_ref[:] + s2b_ref[:])
    final = score * knn_f3
    f3i = _lrelu(jnp.dot(final, iw_ref[:], preferred_element_type=jnp.float32)
                 + ib_ref[:])
    out_ref[0] = f3i


def _sample3d_kernel(uvt_ref, f2pm_ref, mw_ref, mb_ref, a1w_ref, a1b_ref,
                     out_ref):
    c = pl.program_id(1)
    base = c * _P
    x = uvt_ref[0, pl.ds(base, _P), 0:1]
    y = uvt_ref[0, pl.ds(base, _P), 1:2]
    x0f = jnp.floor(x)
    y0f = jnp.floor(y)
    wx1 = x - x0f
    wy1 = y - y0f
    wx0 = 1.0 - wx1
    wy0 = 1.0 - wy1
    x0 = jnp.clip(x0f, 0, _W - 1).astype(jnp.int32)
    x1 = jnp.clip(x0f + 1.0, 0, _W - 1).astype(jnp.int32)
    y0 = jnp.clip(y0f, 0, _H - 1).astype(jnp.int32)
    y1 = jnp.clip(y0f + 1.0, 0, _H - 1).astype(jnp.int32)
    i00 = y0 * _W + x0
    i01 = y0 * _W + x1
    i10 = y1 * _W + x0
    i11 = y1 * _W + x1
    iop = jax.lax.broadcasted_iota(jnp.int32, (_P, _HW), 1)
    zero = jnp.zeros((), jnp.float32)
    smat = (jnp.where(iop == i00, wy0 * wx0, zero)
            + jnp.where(iop == i01, wy0 * wx1, zero)
            + jnp.where(iop == i10, wy1 * wx0, zero)
            + jnp.where(iop == i11, wy1 * wx1, zero))
    f2s = jnp.dot(smat, f2pm_ref[0], preferred_element_type=jnp.float32)
    m = _lrelu(jnp.dot(f2s, mw_ref[:], preferred_element_type=jnp.float32)
               + mb_ref[:])
    a3 = _lrelu(jnp.dot(m, a1w_ref[:], preferred_element_type=jnp.float32)
                + a1b_ref[:])
    out_ref[0] = a3


def _fuse_kernel(f2pm_ref, f3ipm_ref, a3pm_ref, f3pm_ref,
                 f2a1w_ref, f2a1b_ref, f2a2w_ref, f2a2b_ref,
                 f2midw_ref, f2evw_ref, f2odw_ref,
                 f3a2w_ref, f3a2b_ref, f3midw_ref, f3evw_ref, f3odw_ref,
                 out2d_ref, out3d_ref):
    f32 = jnp.float32
    a2 = _lrelu(jnp.dot(f2pm_ref[0], f2a1w_ref[:], preferred_element_type=f32)
                + f2a1b_ref[:])
    b2 = _lrelu(jnp.dot(f3ipm_ref[0], f2a2w_ref[:], preferred_element_type=f32)
                + f2a2b_ref[:])
    w = jnp.sum(a2 + b2, axis=0, keepdims=True) * (1.0 / _HW)
    r = jnp.maximum(jnp.dot(w, f2midw_ref[:], preferred_element_type=f32), 0.0)
    v0 = jax.nn.sigmoid(jnp.dot(r, f2evw_ref[:], preferred_element_type=f32))
    v1 = jax.nn.sigmoid(jnp.dot(r, f2odw_ref[:], preferred_element_type=f32))
    p0, p1 = _pair_softmax(v0, v1)
    out2d_ref[0] = a2 * p0 + b2 * p1

    a3 = a3pm_ref[0]
    b3 = _lrelu(jnp.dot(f3pm_ref[0], f3a2w_ref[:], preferred_element_type=f32)
                + f3a2b_ref[:])
    w3 = jnp.sum(a3 + b3, axis=0, keepdims=True) * (1.0 / _N)
    r3 = jnp.maximum(jnp.dot(w3, f3midw_ref[:], preferred_element_type=f32),
                     0.0)
    v30 = jax.nn.sigmoid(jnp.dot(r3, f3evw_ref[:], preferred_element_type=f32))
    v31 = jax.nn.sigmoid(jnp.dot(r3, f3odw_ref[:], preferred_element_type=f32))
    p30, p31 = _pair_softmax(v30, v31)
    out3d_ref[0] = a3 * p30 + b3 * p31


def _vspec(shape):
    nd = len(shape)
    return pl.BlockSpec(shape, lambda *_: (0,) * nd)


def kernel(uv, feat_2d, feat_3d, interp_out_W, interp_out_b, score1_W,
           score1_b, score2_W, score2_b, mlps3d_W, mlps3d_b, f2_a1_W, f2_a1_b,
           f2_a2_W, f2_a2_b, f2_mid_W, f2_out_W, f3_a1_W, f3_a1_b, f3_a2_W,
           f3_a2_b, f3_mid_W, f3_out_W):
    f32 = jnp.float32
    f2pm = feat_2d.reshape(_BS, _C2D, _HW).transpose(0, 2, 1)
    f3pm = feat_3d.transpose(0, 2, 1)
    uvt = uv.transpose(0, 2, 1)
    cat = jnp.concatenate([uvt, f3pm], axis=2)
    s1 = jnp.concatenate([score1_W.reshape(1, 3),
                          score1_b.reshape(1, 1)], axis=1)
    s2w = score2_W.reshape(1, _C3D)
    s2b = score2_b.reshape(1, _C3D)

    f3i_pm = pl.pallas_call(
        _interp_kernel,
        grid=(_BS, _NC),
        in_specs=[
            pl.BlockSpec(memory_space=pltpu.SMEM),
            pl.BlockSpec((1, 2, _N), lambda b, c: (b, 0, 0)),
            pl.BlockSpec((1, _N, 2 + _C3D), lambda b, c: (b, 0, 0)),
            pl.BlockSpec((1, _C3D), lambda b, c: (0, 0)),
            pl.BlockSpec((1, _C3D), lambda b, c: (0, 0)),
            pl.BlockSpec((_C3D, _C3D), lambda b, c: (0, 0)),
            pl.BlockSpec((1, _C3D), lambda b, c: (0, 0)),
        ],
        out_specs=pl.BlockSpec((1, _P, _C3D), lambda b, c: (b, c, 0)),
        out_shape=jax.ShapeDtypeStruct((_BS, _HW, _C3D), f32),
    )(s1, uv, cat, s2w, s2b, interp_out_W.T, interp_out_b.reshape(1, _C3D))

    a3_pm = pl.pallas_call(
        _sample3d_kernel,
        grid=(_BS, _NC),
        in_specs=[
            pl.BlockSpec((1, _N, 2), lambda b, c: (b, 0, 0)),
            pl.BlockSpec((1, _HW, _C2D), lambda b, c: (b, 0, 0)),
            pl.BlockSpec((_C2D, _C2D), lambda b, c: (0, 0)),
            pl.BlockSpec((1, _C2D), lambda b, c: (0, 0)),
            pl.BlockSpec((_C2D, _C3D), lambda b, c: (0, 0)),
            pl.BlockSpec((1, _C3D), lambda b, c: (0, 0)),
        ],
        out_specs=pl.BlockSpec((1, _P, _C3D), lambda b, c: (b, c, 0)),
        out_shape=jax.ShapeDtypeStruct((_BS, _N, _C3D), f32),
    )(uvt, f2pm, mlps3d_W.T, mlps3d_b.reshape(1, _C2D),
      f3_a1_W.T, f3_a1_b.reshape(1, _C3D))

    out2d_pm, out3d_pm = pl.pallas_call(
        _fuse_kernel,
        grid=(_BS,),
        in_specs=[
            pl.BlockSpec((1, _HW, _C2D), lambda b: (b, 0, 0)),
            pl.BlockSpec((1, _HW, _C3D), lambda b: (b, 0, 0)),
            pl.BlockSpec((1, _N, _C3D), lambda b: (b, 0, 0)),
            pl.BlockSpec((1, _N, _C3D), lambda b: (b, 0, 0)),
            _vspec((_C2D, _C2D)), _vspec((1, _C2D)),
            _vspec((_C3D, _C2D)), _vspec((1, _C2D)),
            _vspec((_C2D, _C2D // 2)),
            _vspec((_C2D // 2, _C2D)), _vspec((_C2D // 2, _C2D)),
            _vspec((_C3D, _C3D)), _vspec((1, _C3D)),
            _vspec((_C3D, _C3D // 2)),
            _vspec((_C3D // 2, _C3D)), _vspec((_C3D // 2, _C3D)),
        ],
        out_specs=[
            pl.BlockSpec((1, _HW, _C2D), lambda b: (b, 0, 0)),
            pl.BlockSpec((1, _N, _C3D), lambda b: (b, 0, 0)),
        ],
        out_shape=[
            jax.ShapeDtypeStruct((_BS, _HW, _C2D), f32),
            jax.ShapeDtypeStruct((_BS, _N, _C3D), f32),
        ],
    )(f2pm, f3i_pm, a3_pm, f3pm,
      f2_a1_W.T, f2_a1_b.reshape(1, _C2D),
      f2_a2_W.T, f2_a2_b.reshape(1, _C2D),
      f2_mid_W.T, f2_out_W[0::2].T, f2_out_W[1::2].T,
      f3_a2_W.T, f3_a2_b.reshape(1, _C3D),
      f3_mid_W.T, f3_out_W[0::2].T, f3_out_W[1::2].T)

    out2d = out2d_pm.transpose(0, 2, 1).reshape(_BS, _C2D, _H, _W)
    out3d = out3d_pm.transpose(0, 2, 1)
    return (out2d, out3d)


# speedup vs baseline: 39.6824x; 1.0393x over previous
import jax
import jax.numpy as jnp
from jax.experimental import pallas as pl
from jax.experimental.pallas import tpu as pltpu

_BS, _C2D, _C3D, _H, _W, _N = 2, 64, 64, 64, 64, 4096
_HW = _H * _W
_P = 512
_NC = _HW // _P


def _lrelu(x):
    return jnp.where(x >= 0, x, 0.1 * x)


def _pair_softmax(v0, v1):
    m = jnp.maximum(v0, v1)
    e0 = jnp.exp(v0 - m)
    e1 = jnp.exp(v1 - m)
    inv = 1.0 / (e0 + e1)
    return e0 * inv, e1 * inv


def _interp_kernel(s1_ref, uv_ref, cat_ref, s2w_ref, s2b_ref,
                   iw_ref, ib_ref, out_ref):
    c = pl.program_id(1)
    base = c * _P
    ux = uv_ref[0, 0:1, :]
    uy = uv_ref[0, 1:2, :]
    pix = base + jax.lax.broadcasted_iota(jnp.int32, (_P, 1), 0)
    gx = (pix % _W).astype(jnp.float32)
    gy = (pix // _W).astype(jnp.float32)
    dx = gx - ux
    dy = gy - uy
    dist = dx * dx + dy * dy
    dmin = jnp.min(dist, axis=1, keepdims=True)
    ion = jax.lax.broadcasted_iota(jnp.int32, (_P, _N), 1)
    idx = jnp.min(jnp.where(dist == dmin, ion, _N), axis=1, keepdims=True)
    oh = (ion == idx).astype(jnp.float32)
    g = jnp.dot(oh, cat_ref[0], preferred_element_type=jnp.float32)
    offx = g[:, 0:1] - gx
    offy = g[:, 1:2] - gy
    knn_f3 = g[:, 2:2 + _C3D]
    nrm = jnp.sqrt(offx * offx + offy * offy)
    s = _lrelu(offx * s1_ref[0, 0] + offy * s1_ref[0, 1]
               + nrm * s1_ref[0, 2] + s1_ref[0, 3])
    score = jax.nn.sigmoid(s * s2w_ref[:] + s2b_ref[:])
    final = score * knn_f3
    f3i = _lrelu(jnp.dot(final, iw_ref[:], preferred_element_type=jnp.float32)
                 + ib_ref[:])
    out_ref[0] = f3i


def _sample3d_kernel(uvt_ref, fyxc_ref, mw_ref, mb_ref, a1w_ref,
                     a1b_ref, out_ref):
    c = pl.program_id(1)
    base = c * _P
    x = uvt_ref[0, pl.ds(base, _P), 0:1]
    y = uvt_ref[0, pl.ds(base, _P), 1:2]
    x0f = jnp.floor(x)
    y0f = jnp.floor(y)
    wx1 = x - x0f
    wy1 = y - y0f
    wx0 = 1.0 - wx1
    wy0 = 1.0 - wy1
    x0 = jnp.clip(x0f, 0, _W - 1).astype(jnp.int32)
    x1 = jnp.clip(x0f + 1.0, 0, _W - 1).astype(jnp.int32)
    y0 = jnp.clip(y0f, 0, _H - 1).astype(jnp.int32)
    y1 = jnp.clip(y0f + 1.0, 0, _H - 1).astype(jnp.int32)
    zero = jnp.zeros((), jnp.float32)
    ioy = jax.lax.broadcasted_iota(jnp.int32, (_P, _H), 1)
    arow = (jnp.where(ioy == y0, wy0, zero)
            + jnp.where(ioy == y1, wy1, zero))
    gy = jnp.dot(arow, fyxc_ref[0], preferred_element_type=jnp.float32)
    iox = jax.lax.broadcasted_iota(jnp.int32, (_P, _W * _C2D), 1) // _C2D
    bx = (jnp.where(iox == x0, wx0, zero)
          + jnp.where(iox == x1, wx1, zero))
    ioe_r = jax.lax.broadcasted_iota(jnp.int32, (_W * _C2D, _C2D), 0)
    ioe_c = jax.lax.broadcasted_iota(jnp.int32, (_W * _C2D, _C2D), 1)
    eye = (ioe_r % _C2D == ioe_c).astype(jnp.float32)
    f2s = jnp.dot(gy * bx, eye, preferred_element_type=jnp.float32)
    m = _lrelu(jnp.dot(f2s, mw_ref[:], preferred_element_type=jnp.float32)
               + mb_ref[:])
    a3 = _lrelu(jnp.dot(m, a1w_ref[:], preferred_element_type=jnp.float32)
                + a1b_ref[:])
    out_ref[0] = a3


def _fuse_kernel(f2pm_ref, f3ipm_ref, a3pm_ref, f3pm_ref,
                 f2a1w_ref, f2a1b_ref, f2a2w_ref, f2a2b_ref,
                 f2midw_ref, f2evw_ref, f2odw_ref,
                 f3a2w_ref, f3a2b_ref, f3midw_ref, f3evw_ref, f3odw_ref,
                 out2d_ref, out3d_ref):
    f32 = jnp.float32
    a2 = _lrelu(jnp.dot(f2pm_ref[0], f2a1w_ref[:], preferred_element_type=f32)
                + f2a1b_ref[:])
    b2 = _lrelu(jnp.dot(f3ipm_ref[0], f2a2w_ref[:], preferred_element_type=f32)
                + f2a2b_ref[:])
    w = jnp.sum(a2 + b2, axis=0, keepdims=True) * (1.0 / _HW)
    r = jnp.maximum(jnp.dot(w, f2midw_ref[:], preferred_element_type=f32), 0.0)
    v0 = jax.nn.sigmoid(jnp.dot(r, f2evw_ref[:], preferred_element_type=f32))
    v1 = jax.nn.sigmoid(jnp.dot(r, f2odw_ref[:], preferred_element_type=f32))
    p0, p1 = _pair_softmax(v0, v1)
    out2d_ref[0] = a2 * p0 + b2 * p1

    a3 = a3pm_ref[0]
    b3 = _lrelu(jnp.dot(f3pm_ref[0], f3a2w_ref[:], preferred_element_type=f32)
                + f3a2b_ref[:])
    w3 = jnp.sum(a3 + b3, axis=0, keepdims=True) * (1.0 / _N)
    r3 = jnp.maximum(jnp.dot(w3, f3midw_ref[:], preferred_element_type=f32),
                     0.0)
    v30 = jax.nn.sigmoid(jnp.dot(r3, f3evw_ref[:], preferred_element_type=f32))
    v31 = jax.nn.sigmoid(jnp.dot(r3, f3odw_ref[:], preferred_element_type=f32))
    p30, p31 = _pair_softmax(v30, v31)
    out3d_ref[0] = a3 * p30 + b3 * p31


def _vspec(shape):
    nd = len(shape)
    return pl.BlockSpec(shape, lambda *_: (0,) * nd)


def kernel(uv, feat_2d, feat_3d, interp_out_W, interp_out_b, score1_W,
           score1_b, score2_W, score2_b, mlps3d_W, mlps3d_b, f2_a1_W, f2_a1_b,
           f2_a2_W, f2_a2_b, f2_mid_W, f2_out_W, f3_a1_W, f3_a1_b, f3_a2_W,
           f3_a2_b, f3_mid_W, f3_out_W):
    f32 = jnp.float32
    f2t = feat_2d.transpose(0, 2, 3, 1)
    f2pm = f2t.reshape(_BS, _HW, _C2D)
    fyxc = f2t.reshape(_BS, _H, _W * _C2D)
    f3pm = feat_3d.transpose(0, 2, 1)
    uvt = uv.transpose(0, 2, 1)
    cat = jnp.concatenate([uvt, f3pm], axis=2)
    s1 = jnp.concatenate([score1_W.reshape(1, 3),
                          score1_b.reshape(1, 1)], axis=1)
    s2w = score2_W.reshape(1, _C3D)
    s2b = score2_b.reshape(1, _C3D)

    f3i_pm = pl.pallas_call(
        _interp_kernel,
        grid=(_BS, _NC),
        in_specs=[
            pl.BlockSpec(memory_space=pltpu.SMEM),
            pl.BlockSpec((1, 2, _N), lambda b, c: (b, 0, 0)),
            pl.BlockSpec((1, _N, 2 + _C3D), lambda b, c: (b, 0, 0)),
            pl.BlockSpec((1, _C3D), lambda b, c: (0, 0)),
            pl.BlockSpec((1, _C3D), lambda b, c: (0, 0)),
            pl.BlockSpec((_C3D, _C3D), lambda b, c: (0, 0)),
            pl.BlockSpec((1, _C3D), lambda b, c: (0, 0)),
        ],
        out_specs=pl.BlockSpec((1, _P, _C3D), lambda b, c: (b, c, 0)),
        out_shape=jax.ShapeDtypeStruct((_BS, _HW, _C3D), f32),
    )(s1, uv, cat, s2w, s2b, interp_out_W.T,
      interp_out_b.reshape(1, _C3D))

    a3_pm = pl.pallas_call(
        _sample3d_kernel,
        grid=(_BS, _NC),
        in_specs=[
            pl.BlockSpec((1, _N, 2), lambda b, c: (b, 0, 0)),
            pl.BlockSpec((1, _H, _W * _C2D), lambda b, c: (b, 0, 0)),
            pl.BlockSpec((_C2D, _C2D), lambda b, c: (0, 0)),
            pl.BlockSpec((1, _C2D), lambda b, c: (0, 0)),
            pl.BlockSpec((_C2D, _C3D), lambda b, c: (0, 0)),
            pl.BlockSpec((1, _C3D), lambda b, c: (0, 0)),
        ],
        out_specs=pl.BlockSpec((1, _P, _C3D), lambda b, c: (b, c, 0)),
        out_shape=jax.ShapeDtypeStruct((_BS, _N, _C3D), f32),
    )(uvt, fyxc, mlps3d_W.T, mlps3d_b.reshape(1, _C2D),
      f3_a1_W.T, f3_a1_b.reshape(1, _C3D))

    out2d_pm, out3d_pm = pl.pallas_call(
        _fuse_kernel,
        grid=(_BS,),
        in_specs=[
            pl.BlockSpec((1, _HW, _C2D), lambda b: (b, 0, 0)),
            pl.BlockSpec((1, _HW, _C3D), lambda b: (b, 0, 0)),
            pl.BlockSpec((1, _N, _C3D), lambda b: (b, 0, 0)),
            pl.BlockSpec((1, _N, _C3D), lambda b: (b, 0, 0)),
            _vspec((_C2D, _C2D)), _vspec((1, _C2D)),
            _vspec((_C3D, _C2D)), _vspec((1, _C2D)),
            _vspec((_C2D, _C2D // 2)),
            _vspec((_C2D // 2, _C2D)), _vspec((_C2D // 2, _C2D)),
            _vspec((_C3D, _C3D)), _vspec((1, _C3D)),
            _vspec((_C3D, _C3D // 2)),
            _vspec((_C3D // 2, _C3D)), _vspec((_C3D // 2, _C3D)),
        ],
        out_specs=[
            pl.BlockSpec((1, _HW, _C2D), lambda b: (b, 0, 0)),
            pl.BlockSpec((1, _N, _C3D), lambda b: (b, 0, 0)),
        ],
        out_shape=[
            jax.ShapeDtypeStruct((_BS, _HW, _C2D), f32),
            jax.ShapeDtypeStruct((_BS, _N, _C3D), f32),
        ],
    )(f2pm, f3i_pm, a3_pm, f3pm,
      f2_a1_W.T, f2_a1_b.reshape(1, _C2D),
      f2_a2_W.T, f2_a2_b.reshape(1, _C2D),
      f2_mid_W.T, f2_out_W[0::2].T, f2_out_W[1::2].T,
      f3_a2_W.T, f3_a2_b.reshape(1, _C3D),
      f3_mid_W.T, f3_out_W[0::2].T, f3_out_W[1::2].T)

    out2d = out2d_pm.transpose(0, 2, 1).reshape(_BS, _C2D, _H, _W)
    out3d = out3d_pm.transpose(0, 2, 1)
    return (out2d, out3d)


# speedup vs baseline: 51.2171x; 1.2907x over previous
import functools

import jax
import jax.numpy as jnp
from jax import lax
from jax.experimental import pallas as pl
from jax.experimental.pallas import tpu as pltpu
from jax.experimental.pallas import tpu_sc as plsc

_BS, _C2D, _C3D, _H, _W, _N = 2, 64, 64, 64, 64, 4096
_HW = _H * _W
_P = 512
_NC = _HW // _P
_NWORK = 32
_PPW = (_BS * _N) // _NWORK
_LG = _PPW // 16


def _lrelu(x):
    return jnp.where(x >= 0, x, 0.1 * x)


def _pair_softmax(v0, v1):
    m = jnp.maximum(v0, v1)
    e0 = jnp.exp(v0 - m)
    e1 = jnp.exp(v1 - m)
    inv = 1.0 / (e0 + e1)
    return e0 * inv, e1 * inv


def _interp_kernel(s1_ref, uv_ref, cat_ref, s2w_ref, s2b_ref,
                   iw_ref, ib_ref, out_ref):
    c = pl.program_id(1)
    base = c * _P
    ux = uv_ref[0, 0:1, :]
    uy = uv_ref[0, 1:2, :]
    pix = base + jax.lax.broadcasted_iota(jnp.int32, (_P, 1), 0)
    gx = (pix % _W).astype(jnp.float32)
    gy = (pix // _W).astype(jnp.float32)
    dx = gx - ux
    dy = gy - uy
    dist = dx * dx + dy * dy
    dmin = jnp.min(dist, axis=1, keepdims=True)
    ion = jax.lax.broadcasted_iota(jnp.int32, (_P, _N), 1)
    idx = jnp.min(jnp.where(dist == dmin, ion, _N), axis=1, keepdims=True)
    oh = (ion == idx).astype(jnp.float32)
    g = jnp.dot(oh, cat_ref[0], preferred_element_type=jnp.float32)
    offx = g[:, 0:1] - gx
    offy = g[:, 1:2] - gy
    knn_f3 = g[:, 2:2 + _C3D]
    nrm = jnp.sqrt(offx * offx + offy * offy)
    s = _lrelu(offx * s1_ref[0, 0] + offy * s1_ref[0, 1]
               + nrm * s1_ref[0, 2] + s1_ref[0, 3])
    score = jax.nn.sigmoid(s * s2w_ref[:] + s2b_ref[:])
    final = score * knn_f3
    f3i = _lrelu(jnp.dot(final, iw_ref[:], preferred_element_type=jnp.float32)
                 + ib_ref[:])
    out_ref[0] = f3i


def _sc_bilinear_kernel(ux_hbm, uy_hbm, tab_hbm, out_hbm,
                        xv, yv, w00v, w01v, w10v, w11v,
                        i00a, i01a, i10a, i11a, i00b, i01b, i10b, i11b,
                        r00, r01, r10, r11, outv, sem):
    wid = lax.axis_index("s") * 2 + lax.axis_index("c")
    base = wid * _PPW
    rbase = (base // _N) * _N
    pltpu.sync_copy(ux_hbm.at[pl.ds(base, _PPW)], xv)
    pltpu.sync_copy(uy_hbm.at[pl.ds(base, _PPW)], yv)

    idx_half = ((i00a, i01a, i10a, i11a), (i00b, i01b, i10b, i11b))
    for g in range(_LG):
        sl = pl.ds(g * 16, 16)
        x = xv[sl]
        y = yv[sl]
        x0i = x.astype(jnp.int32)
        y0i = y.astype(jnp.int32)
        wx1 = x - x0i.astype(jnp.float32)
        wy1 = y - y0i.astype(jnp.float32)
        wx0 = 1.0 - wx1
        wy0 = 1.0 - wy1
        w00v[sl] = wy0 * wx0
        w01v[sl] = wy0 * wx1
        w10v[sl] = wy1 * wx0
        w11v[sl] = wy1 * wx1
        x0 = jnp.minimum(jnp.maximum(x0i, 0), _W - 1)
        x1 = jnp.minimum(jnp.maximum(x0i + 1, 0), _W - 1)
        y0 = jnp.minimum(jnp.maximum(y0i, 0), _H - 1)
        y1 = jnp.minimum(jnp.maximum(y0i + 1, 0), _H - 1)
        i00h, i01h, i10h, i11h = idx_half[g // (_LG // 2)]
        hsl = pl.ds((g % (_LG // 2)) * 16, 16)
        i00h[hsl] = y0 * _W + x0 + rbase
        i01h[hsl] = y0 * _W + x1 + rbase
        i10h[hsl] = y1 * _W + x0 + rbase
        i11h[hsl] = y1 * _W + x1 + rbase

    hp = _PPW // 2
    for h in range(2):
        i00h, i01h, i10h, i11h = idx_half[h]
        c0 = pltpu.async_copy(tab_hbm.at[i00h], r00, sem)
        c1 = pltpu.async_copy(tab_hbm.at[i01h], r01, sem)
        c2 = pltpu.async_copy(tab_hbm.at[i10h], r10, sem)
        c3 = pltpu.async_copy(tab_hbm.at[i11h], r11, sem)
        c0.wait()
        c1.wait()
        c2.wait()
        c3.wait()

        def blend_body(g2, _, h=h):
            wsl = pl.ds(h * hp + g2 * 16, 16)
            wv00 = w00v[wsl]
            wv01 = w01v[wsl]
            wv10 = w10v[wsl]
            wv11 = w11v[wsl]
            for k in range(16):
                j = g2 * 16 + k
                p = h * hp + j
                for cg in range(_C2D // 16):
                    cs = pl.ds(cg * 16, 16)
                    outv[p, cs] = (wv00[k] * r00[j, cs]
                                   + wv01[k] * r01[j, cs]
                                   + wv10[k] * r10[j, cs]
                                   + wv11[k] * r11[j, cs])
            return _

        lax.fori_loop(0, hp // 16, blend_body, 0)

    pltpu.sync_copy(outv, out_hbm.at[pl.ds(base, _PPW)])


def _fuse_kernel(f2pm_ref, f3ipm_ref, f2spm_ref, f3pm_ref,
                 mw_ref, mb_ref, a1w_ref, a1b_ref,
                 f2a1w_ref, f2a1b_ref, f2a2w_ref, f2a2b_ref,
                 f2midw_ref, f2evw_ref, f2odw_ref,
                 f3a2w_ref, f3a2b_ref, f3midw_ref, f3evw_ref, f3odw_ref,
                 out2d_ref, out3d_ref):
    f32 = jnp.float32
    a2 = _lrelu(jnp.dot(f2pm_ref[0], f2a1w_ref[:], preferred_element_type=f32)
                + f2a1b_ref[:])
    b2 = _lrelu(jnp.dot(f3ipm_ref[0], f2a2w_ref[:], preferred_element_type=f32)
                + f2a2b_ref[:])
    w = jnp.sum(a2 + b2, axis=0, keepdims=True) * (1.0 / _HW)
    r = jnp.maximum(jnp.dot(w, f2midw_ref[:], preferred_element_type=f32), 0.0)
    v0 = jax.nn.sigmoid(jnp.dot(r, f2evw_ref[:], preferred_element_type=f32))
    v1 = jax.nn.sigmoid(jnp.dot(r, f2odw_ref[:], preferred_element_type=f32))
    p0, p1 = _pair_softmax(v0, v1)
    out2d_ref[0] = a2 * p0 + b2 * p1

    m = _lrelu(jnp.dot(f2spm_ref[0], mw_ref[:], preferred_element_type=f32)
               + mb_ref[:])
    a3 = _lrelu(jnp.dot(m, a1w_ref[:], preferred_element_type=f32)
                + a1b_ref[:])
    b3 = _lrelu(jnp.dot(f3pm_ref[0], f3a2w_ref[:], preferred_element_type=f32)
                + f3a2b_ref[:])
    w3 = jnp.sum(a3 + b3, axis=0, keepdims=True) * (1.0 / _N)
    r3 = jnp.maximum(jnp.dot(w3, f3midw_ref[:], preferred_element_type=f32),
                     0.0)
    v30 = jax.nn.sigmoid(jnp.dot(r3, f3evw_ref[:], preferred_element_type=f32))
    v31 = jax.nn.sigmoid(jnp.dot(r3, f3odw_ref[:], preferred_element_type=f32))
    p30, p31 = _pair_softmax(v30, v31)
    out3d_ref[0] = a3 * p30 + b3 * p31


def _vspec(shape):
    nd = len(shape)
    return pl.BlockSpec(shape, lambda *_: (0,) * nd)


def kernel(uv, feat_2d, feat_3d, interp_out_W, interp_out_b, score1_W,
           score1_b, score2_W, score2_b, mlps3d_W, mlps3d_b, f2_a1_W, f2_a1_b,
           f2_a2_W, f2_a2_b, f2_mid_W, f2_out_W, f3_a1_W, f3_a1_b, f3_a2_W,
           f3_a2_b, f3_mid_W, f3_out_W):
    f32 = jnp.float32
    f2t = feat_2d.transpose(0, 2, 3, 1)
    f2pm = f2t.reshape(_BS, _HW, _C2D)
    f3pm = feat_3d.transpose(0, 2, 1)
    uvt = uv.transpose(0, 2, 1)
    cat = jnp.concatenate([uvt, f3pm], axis=2)
    s1 = jnp.concatenate([score1_W.reshape(1, 3),
                          score1_b.reshape(1, 1)], axis=1)
    s2w = score2_W.reshape(1, _C3D)
    s2b = score2_b.reshape(1, _C3D)

    f3i_pm = pl.pallas_call(
        _interp_kernel,
        grid=(_BS, _NC),
        in_specs=[
            pl.BlockSpec(memory_space=pltpu.SMEM),
            pl.BlockSpec((1, 2, _N), lambda b, c: (b, 0, 0)),
            pl.BlockSpec((1, _N, 2 + _C3D), lambda b, c: (b, 0, 0)),
            pl.BlockSpec((1, _C3D), lambda b, c: (0, 0)),
            pl.BlockSpec((1, _C3D), lambda b, c: (0, 0)),
            pl.BlockSpec((_C3D, _C3D), lambda b, c: (0, 0)),
            pl.BlockSpec((1, _C3D), lambda b, c: (0, 0)),
        ],
        out_specs=pl.BlockSpec((1, _P, _C3D), lambda b, c: (b, c, 0)),
        out_shape=jax.ShapeDtypeStruct((_BS, _HW, _C3D), f32),
    )(s1, uv, cat, s2w, s2b, interp_out_W.T,
      interp_out_b.reshape(1, _C3D))

    sc_bilinear = functools.partial(
        pl.kernel,
        mesh=plsc.VectorSubcoreMesh(core_axis_name="c", subcore_axis_name="s"),
        out_type=jax.ShapeDtypeStruct((_BS * _N, _C2D), f32),
        scratch_types=[
            pltpu.VMEM((_PPW,), f32), pltpu.VMEM((_PPW,), f32),
            pltpu.VMEM((_PPW,), f32), pltpu.VMEM((_PPW,), f32),
            pltpu.VMEM((_PPW,), f32), pltpu.VMEM((_PPW,), f32),
            pltpu.VMEM((_PPW // 2,), jnp.int32),
            pltpu.VMEM((_PPW // 2,), jnp.int32),
            pltpu.VMEM((_PPW // 2,), jnp.int32),
            pltpu.VMEM((_PPW // 2,), jnp.int32),
            pltpu.VMEM((_PPW // 2,), jnp.int32),
            pltpu.VMEM((_PPW // 2,), jnp.int32),
            pltpu.VMEM((_PPW // 2,), jnp.int32),
            pltpu.VMEM((_PPW // 2,), jnp.int32),
            pltpu.VMEM((_PPW // 2, 2 * _C2D), f32),
            pltpu.VMEM((_PPW // 2, 2 * _C2D), f32),
            pltpu.VMEM((_PPW // 2, 2 * _C2D), f32),
            pltpu.VMEM((_PPW // 2, 2 * _C2D), f32),
            pltpu.VMEM((_PPW, _C2D), f32),
            pltpu.SemaphoreType.DMA,
        ],
    )(_sc_bilinear_kernel)
    ux = uv[:, 0, :].reshape(_BS * _N)
    uy = uv[:, 1, :].reshape(_BS * _N)
    tab = jnp.concatenate(
        [f2t.reshape(_BS * _HW, _C2D),
         jnp.zeros((_BS * _HW, _C2D), f32)], axis=1)
    f2s_pm = sc_bilinear(ux, uy, tab).reshape(_BS, _N, _C2D)

    out2d_pm, out3d_pm = pl.pallas_call(
        _fuse_kernel,
        grid=(_BS,),
        in_specs=[
            pl.BlockSpec((1, _HW, _C2D), lambda b: (b, 0, 0)),
            pl.BlockSpec((1, _HW, _C3D), lambda b: (b, 0, 0)),
            pl.BlockSpec((1, _N, _C2D), lambda b: (b, 0, 0)),
            pl.BlockSpec((1, _N, _C3D), lambda b: (b, 0, 0)),
            _vspec((_C2D, _C2D)), _vspec((1, _C2D)),
            _vspec((_C2D, _C3D)), _vspec((1, _C3D)),
            _vspec((_C2D, _C2D)), _vspec((1, _C2D)),
            _vspec((_C3D, _C2D)), _vspec((1, _C2D)),
            _vspec((_C2D, _C2D // 2)),
            _vspec((_C2D // 2, _C2D)), _vspec((_C2D // 2, _C2D)),
            _vspec((_C3D, _C3D)), _vspec((1, _C3D)),
            _vspec((_C3D, _C3D // 2)),
            _vspec((_C3D // 2, _C3D)), _vspec((_C3D // 2, _C3D)),
        ],
        out_specs=[
            pl.BlockSpec((1, _HW, _C2D), lambda b: (b, 0, 0)),
            pl.BlockSpec((1, _N, _C3D), lambda b: (b, 0, 0)),
        ],
        out_shape=[
            jax.ShapeDtypeStruct((_BS, _HW, _C2D), f32),
            jax.ShapeDtypeStruct((_BS, _N, _C3D), f32),
        ],
    )(f2pm, f3i_pm, f2s_pm, f3pm,
      mlps3d_W.T, mlps3d_b.reshape(1, _C2D),
      f3_a1_W.T, f3_a1_b.reshape(1, _C3D),
      f2_a1_W.T, f2_a1_b.reshape(1, _C2D),
      f2_a2_W.T, f2_a2_b.reshape(1, _C2D),
      f2_mid_W.T, f2_out_W[0::2].T, f2_out_W[1::2].T,
      f3_a2_W.T, f3_a2_b.reshape(1, _C3D),
      f3_mid_W.T, f3_out_W[0::2].T, f3_out_W[1::2].T)

    out2d = out2d_pm.transpose(0, 2, 1).reshape(_BS, _C2D, _H, _W)
    out3d = out3d_pm.transpose(0, 2, 1)
    return (out2d, out3d)


# speedup vs baseline: 52.3805x; 1.0227x over previous
import functools

import jax
import jax.numpy as jnp
from jax import lax
from jax.experimental import pallas as pl
from jax.experimental.pallas import tpu as pltpu
from jax.experimental.pallas import tpu_sc as plsc

_BS, _C2D, _C3D, _H, _W, _N = 2, 64, 64, 64, 64, 4096
_HW = _H * _W
_P = 1024
_NC = _HW // _P
_NWORK = 32
_PPW = (_BS * _N) // _NWORK
_LG = _PPW // 16


def _lrelu(x):
    return jnp.where(x >= 0, x, 0.1 * x)


def _pair_softmax(v0, v1):
    m = jnp.maximum(v0, v1)
    e0 = jnp.exp(v0 - m)
    e1 = jnp.exp(v1 - m)
    inv = 1.0 / (e0 + e1)
    return e0 * inv, e1 * inv


def _interp_kernel(s1_ref, uv_ref, cat_ref, s2w_ref, s2b_ref,
                   iw_ref, ib_ref, out_ref):
    c = pl.program_id(1)
    base = c * _P
    ux = uv_ref[0, 0:1, :]
    uy = uv_ref[0, 1:2, :]
    pix = base + jax.lax.broadcasted_iota(jnp.int32, (_P, 1), 0)
    gx = (pix % _W).astype(jnp.float32)
    gy = (pix // _W).astype(jnp.float32)
    dx = gx - ux
    dy = gy - uy
    dist = dx * dx + dy * dy
    dmin = jnp.min(dist, axis=1, keepdims=True)
    ion = jax.lax.broadcasted_iota(jnp.int32, (_P, _N), 1)
    idx = jnp.min(jnp.where(dist == dmin, ion, _N), axis=1, keepdims=True)
    oh = (ion == idx).astype(jnp.float32)
    g = jnp.dot(oh, cat_ref[0], preferred_element_type=jnp.float32)
    offx = g[:, 0:1] - gx
    offy = g[:, 1:2] - gy
    knn_f3 = g[:, 2:2 + _C3D]
    nrm = jnp.sqrt(offx * offx + offy * offy)
    s = _lrelu(offx * s1_ref[0, 0] + offy * s1_ref[0, 1]
               + nrm * s1_ref[0, 2] + s1_ref[0, 3])
    score = jax.nn.sigmoid(s * s2w_ref[:] + s2b_ref[:])
    final = score * knn_f3
    f3i = _lrelu(jnp.dot(final, iw_ref[:], preferred_element_type=jnp.float32)
                 + ib_ref[:])
    out_ref[0] = f3i


def _sc_bilinear_kernel(ux_hbm, uy_hbm, tab_hbm, out_hbm,
                        xv, yv, w00v, w01v, w10v, w11v,
                        i00a, i01a, i10a, i11a, i00b, i01b, i10b, i11b,
                        r00, r01, r10, r11, outv, sem):
    wid = lax.axis_index("s") * 2 + lax.axis_index("c")
    base = wid * _PPW
    rbase = (base // _N) * _N
    pltpu.sync_copy(ux_hbm.at[pl.ds(base, _PPW)], xv)
    pltpu.sync_copy(uy_hbm.at[pl.ds(base, _PPW)], yv)

    idx_half = ((i00a, i01a, i10a, i11a), (i00b, i01b, i10b, i11b))
    for g in range(_LG):
        sl = pl.ds(g * 16, 16)
        x = xv[sl]
        y = yv[sl]
        x0i = x.astype(jnp.int32)
        y0i = y.astype(jnp.int32)
        wx1 = x - x0i.astype(jnp.float32)
        wy1 = y - y0i.astype(jnp.float32)
        wx0 = 1.0 - wx1
        wy0 = 1.0 - wy1
        w00v[sl] = wy0 * wx0
        w01v[sl] = wy0 * wx1
        w10v[sl] = wy1 * wx0
        w11v[sl] = wy1 * wx1
        x0 = jnp.minimum(jnp.maximum(x0i, 0), _W - 1)
        x1 = jnp.minimum(jnp.maximum(x0i + 1, 0), _W - 1)
        y0 = jnp.minimum(jnp.maximum(y0i, 0), _H - 1)
        y1 = jnp.minimum(jnp.maximum(y0i + 1, 0), _H - 1)
        i00h, i01h, i10h, i11h = idx_half[g // (_LG // 2)]
        hsl = pl.ds((g % (_LG // 2)) * 16, 16)
        i00h[hsl] = y0 * _W + x0 + rbase
        i01h[hsl] = y0 * _W + x1 + rbase
        i10h[hsl] = y1 * _W + x0 + rbase
        i11h[hsl] = y1 * _W + x1 + rbase

    hp = _PPW // 2
    for h in range(2):
        i00h, i01h, i10h, i11h = idx_half[h]
        c0 = pltpu.async_copy(tab_hbm.at[i00h], r00, sem)
        c1 = pltpu.async_copy(tab_hbm.at[i01h], r01, sem)
        c2 = pltpu.async_copy(tab_hbm.at[i10h], r10, sem)
        c3 = pltpu.async_copy(tab_hbm.at[i11h], r11, sem)
        c0.wait()
        c1.wait()
        c2.wait()
        c3.wait()

        def blend_body(g2, _, h=h):
            wsl = pl.ds(h * hp + g2 * 16, 16)
            wv00 = w00v[wsl]
            wv01 = w01v[wsl]
            wv10 = w10v[wsl]
            wv11 = w11v[wsl]
            for k in range(16):
                j = g2 * 16 + k
                p = h * hp + j
                for cg in range(_C2D // 16):
                    cs = pl.ds(cg * 16, 16)
                    outv[p, cs] = (wv00[k] * r00[j, cs]
                                   + wv01[k] * r01[j, cs]
                                   + wv10[k] * r10[j, cs]
                                   + wv11[k] * r11[j, cs])
            return _

        lax.fori_loop(0, hp // 16, blend_body, 0)

    pltpu.sync_copy(outv, out_hbm.at[pl.ds(base, _PPW)])


def _fuse_kernel(f2pm_ref, f3ipm_ref, f2spm_ref, f3pm_ref,
                 mw_ref, mb_ref, a1w_ref, a1b_ref,
                 f2a1w_ref, f2a1b_ref, f2a2w_ref, f2a2b_ref,
                 f2midw_ref, f2evw_ref, f2odw_ref,
                 f3a2w_ref, f3a2b_ref, f3midw_ref, f3evw_ref, f3odw_ref,
                 out2d_ref, out3d_ref):
    f32 = jnp.float32
    a2 = _lrelu(jnp.dot(f2pm_ref[0], f2a1w_ref[:], preferred_element_type=f32)
                + f2a1b_ref[:])
    b2 = _lrelu(jnp.dot(f3ipm_ref[0], f2a2w_ref[:], preferred_element_type=f32)
                + f2a2b_ref[:])
    w = jnp.sum(a2 + b2, axis=0, keepdims=True) * (1.0 / _HW)
    r = jnp.maximum(jnp.dot(w, f2midw_ref[:], preferred_element_type=f32), 0.0)
    v0 = jax.nn.sigmoid(jnp.dot(r, f2evw_ref[:], preferred_element_type=f32))
    v1 = jax.nn.sigmoid(jnp.dot(r, f2odw_ref[:], preferred_element_type=f32))
    p0, p1 = _pair_softmax(v0, v1)
    out2d_ref[0] = a2 * p0 + b2 * p1

    m = _lrelu(jnp.dot(f2spm_ref[0], mw_ref[:], preferred_element_type=f32)
               + mb_ref[:])
    a3 = _lrelu(jnp.dot(m, a1w_ref[:], preferred_element_type=f32)
                + a1b_ref[:])
    b3 = _lrelu(jnp.dot(f3pm_ref[0], f3a2w_ref[:], preferred_element_type=f32)
                + f3a2b_ref[:])
    w3 = jnp.sum(a3 + b3, axis=0, keepdims=True) * (1.0 / _N)
    r3 = jnp.maximum(jnp.dot(w3, f3midw_ref[:], preferred_element_type=f32),
                     0.0)
    v30 = jax.nn.sigmoid(jnp.dot(r3, f3evw_ref[:], preferred_element_type=f32))
    v31 = jax.nn.sigmoid(jnp.dot(r3, f3odw_ref[:], preferred_element_type=f32))
    p30, p31 = _pair_softmax(v30, v31)
    out3d_ref[0] = a3 * p30 + b3 * p31


def _vspec(shape):
    nd = len(shape)
    return pl.BlockSpec(shape, lambda *_: (0,) * nd)


def kernel(uv, feat_2d, feat_3d, interp_out_W, interp_out_b, score1_W,
           score1_b, score2_W, score2_b, mlps3d_W, mlps3d_b, f2_a1_W, f2_a1_b,
           f2_a2_W, f2_a2_b, f2_mid_W, f2_out_W, f3_a1_W, f3_a1_b, f3_a2_W,
           f3_a2_b, f3_mid_W, f3_out_W):
    f32 = jnp.float32
    f2t = feat_2d.transpose(0, 2, 3, 1)
    f2pm = f2t.reshape(_BS, _HW, _C2D)
    f3pm = feat_3d.transpose(0, 2, 1)
    uvt = uv.transpose(0, 2, 1)
    cat = jnp.concatenate([uvt, f3pm], axis=2)
    s1 = jnp.concatenate([score1_W.reshape(1, 3),
                          score1_b.reshape(1, 1)], axis=1)
    s2w = score2_W.reshape(1, _C3D)
    s2b = score2_b.reshape(1, _C3D)

    f3i_pm = pl.pallas_call(
        _interp_kernel,
        grid=(_BS, _NC),
        in_specs=[
            pl.BlockSpec(memory_space=pltpu.SMEM),
            pl.BlockSpec((1, 2, _N), lambda b, c: (b, 0, 0)),
            pl.BlockSpec((1, _N, 2 + _C3D), lambda b, c: (b, 0, 0)),
            pl.BlockSpec((1, _C3D), lambda b, c: (0, 0)),
            pl.BlockSpec((1, _C3D), lambda b, c: (0, 0)),
            pl.BlockSpec((_C3D, _C3D), lambda b, c: (0, 0)),
            pl.BlockSpec((1, _C3D), lambda b, c: (0, 0)),
        ],
        out_specs=pl.BlockSpec((1, _P, _C3D), lambda b, c: (b, c, 0)),
        out_shape=jax.ShapeDtypeStruct((_BS, _HW, _C3D), f32),
    )(s1, uv, cat, s2w, s2b, interp_out_W.T,
      interp_out_b.reshape(1, _C3D))

    sc_bilinear = functools.partial(
        pl.kernel,
        mesh=plsc.VectorSubcoreMesh(core_axis_name="c", subcore_axis_name="s"),
        out_type=jax.ShapeDtypeStruct((_BS * _N, _C2D), f32),
        scratch_types=[
            pltpu.VMEM((_PPW,), f32), pltpu.VMEM((_PPW,), f32),
            pltpu.VMEM((_PPW,), f32), pltpu.VMEM((_PPW,), f32),
            pltpu.VMEM((_PPW,), f32), pltpu.VMEM((_PPW,), f32),
            pltpu.VMEM((_PPW // 2,), jnp.int32),
            pltpu.VMEM((_PPW // 2,), jnp.int32),
            pltpu.VMEM((_PPW // 2,), jnp.int32),
            pltpu.VMEM((_PPW // 2,), jnp.int32),
            pltpu.VMEM((_PPW // 2,), jnp.int32),
            pltpu.VMEM((_PPW // 2,), jnp.int32),
            pltpu.VMEM((_PPW // 2,), jnp.int32),
            pltpu.VMEM((_PPW // 2,), jnp.int32),
            pltpu.VMEM((_PPW // 2, 2 * _C2D), f32),
            pltpu.VMEM((_PPW // 2, 2 * _C2D), f32),
            pltpu.VMEM((_PPW // 2, 2 * _C2D), f32),
            pltpu.VMEM((_PPW // 2, 2 * _C2D), f32),
            pltpu.VMEM((_PPW, _C2D), f32),
            pltpu.SemaphoreType.DMA,
        ],
    )(_sc_bilinear_kernel)
    ux = uv[:, 0, :].reshape(_BS * _N)
    uy = uv[:, 1, :].reshape(_BS * _N)
    tab = jnp.concatenate(
        [f2t.reshape(_BS * _HW, _C2D),
         jnp.zeros((_BS * _HW, _C2D), f32)], axis=1)
    f2s_pm = sc_bilinear(ux, uy, tab).reshape(_BS, _N, _C2D)

    out2d_pm, out3d_pm = pl.pallas_call(
        _fuse_kernel,
        grid=(_BS,),
        in_specs=[
            pl.BlockSpec((1, _HW, _C2D), lambda b: (b, 0, 0)),
            pl.BlockSpec((1, _HW, _C3D), lambda b: (b, 0, 0)),
            pl.BlockSpec((1, _N, _C2D), lambda b: (b, 0, 0)),
            pl.BlockSpec((1, _N, _C3D), lambda b: (b, 0, 0)),
            _vspec((_C2D, _C2D)), _vspec((1, _C2D)),
            _vspec((_C2D, _C3D)), _vspec((1, _C3D)),
            _vspec((_C2D, _C2D)), _vspec((1, _C2D)),
            _vspec((_C3D, _C2D)), _vspec((1, _C2D)),
            _vspec((_C2D, _C2D // 2)),
            _vspec((_C2D // 2, _C2D)), _vspec((_C2D // 2, _C2D)),
            _vspec((_C3D, _C3D)), _vspec((1, _C3D)),
            _vspec((_C3D, _C3D // 2)),
            _vspec((_C3D // 2, _C3D)), _vspec((_C3D // 2, _C3D)),
        ],
        out_specs=[
            pl.BlockSpec((1, _HW, _C2D), lambda b: (b, 0, 0)),
            pl.BlockSpec((1, _N, _C3D), lambda b: (b, 0, 0)),
        ],
        out_shape=[
            jax.ShapeDtypeStruct((_BS, _HW, _C2D), f32),
            jax.ShapeDtypeStruct((_BS, _N, _C3D), f32),
        ],
    )(f2pm, f3i_pm, f2s_pm, f3pm,
      mlps3d_W.T, mlps3d_b.reshape(1, _C2D),
      f3_a1_W.T, f3_a1_b.reshape(1, _C3D),
      f2_a1_W.T, f2_a1_b.reshape(1, _C2D),
      f2_a2_W.T, f2_a2_b.reshape(1, _C2D),
      f2_mid_W.T, f2_out_W[0::2].T, f2_out_W[1::2].T,
      f3_a2_W.T, f3_a2_b.reshape(1, _C3D),
      f3_mid_W.T, f3_out_W[0::2].T, f3_out_W[1::2].T)

    out2d = out2d_pm.transpose(0, 2, 1).reshape(_BS, _C2D, _H, _W)
    out3d = out3d_pm.transpose(0, 2, 1)
    return (out2d, out3d)


# speedup vs baseline: 54.6789x; 1.0439x over previous
import functools

import jax
import jax.numpy as jnp
from jax import lax
from jax.experimental import pallas as pl
from jax.experimental.pallas import tpu as pltpu
from jax.experimental.pallas import tpu_sc as plsc

_BS, _C2D, _C3D, _H, _W, _N = 2, 64, 64, 64, 64, 4096
_HW = _H * _W
_P = 1024
_NC = _HW // _P
_NWORK = 32
_PPW = (_BS * _N) // _NWORK
_LG = _PPW // 16


def _lrelu(x):
    return jnp.where(x >= 0, x, 0.1 * x)


def _pair_softmax(v0, v1):
    m = jnp.maximum(v0, v1)
    e0 = jnp.exp(v0 - m)
    e1 = jnp.exp(v1 - m)
    inv = 1.0 / (e0 + e1)
    return e0 * inv, e1 * inv


def _interp_kernel(s1_ref, uv_ref, cat_ref, s2w_ref, s2b_ref,
                   iw_ref, ib_ref, out_ref):
    c = pl.program_id(1)
    base = c * _P
    ux = uv_ref[0, 0:1, :]
    uy = uv_ref[0, 1:2, :]
    pix = base + jax.lax.broadcasted_iota(jnp.int32, (_P, 1), 0)
    gx = (pix % _W).astype(jnp.float32)
    gy = (pix // _W).astype(jnp.float32)
    dx = gx - ux
    dy = gy - uy
    dist = dx * dx + dy * dy
    dmin = jnp.min(dist, axis=1, keepdims=True)
    ion = jax.lax.broadcasted_iota(jnp.int32, (_P, _N), 1)
    idx = jnp.min(jnp.where(dist == dmin, ion, _N), axis=1, keepdims=True)
    oh = (ion == idx).astype(jnp.float32)
    g = jnp.dot(oh, cat_ref[0], preferred_element_type=jnp.float32)
    offx = g[:, 0:1] - gx
    offy = g[:, 1:2] - gy
    knn_f3 = g[:, 2:2 + _C3D]
    nrm = jnp.sqrt(offx * offx + offy * offy)
    s = _lrelu(offx * s1_ref[0, 0] + offy * s1_ref[0, 1]
               + nrm * s1_ref[0, 2] + s1_ref[0, 3])
    score = jax.nn.sigmoid(s * s2w_ref[:] + s2b_ref[:])
    final = score * knn_f3
    f3i = _lrelu(jnp.dot(final, iw_ref[:], preferred_element_type=jnp.float32)
                 + ib_ref[:])
    out_ref[0] = f3i


def _sc_bilinear_kernel(ux_hbm, uy_hbm, tab_hbm, out_hbm,
                        xv, yv, w00v, w01v, w10v, w11v,
                        i00a, i01a, i10a, i11a, i00b, i01b, i10b, i11b,
                        r00, r01, r10, r11, outv, sem):
    wid = lax.axis_index("s") * 2 + lax.axis_index("c")
    base = wid * _PPW
    rbase = (base // _N) * _N
    pltpu.sync_copy(ux_hbm.at[pl.ds(base, _PPW)], xv)
    pltpu.sync_copy(uy_hbm.at[pl.ds(base, _PPW)], yv)

    idx_half = ((i00a, i01a, i10a, i11a), (i00b, i01b, i10b, i11b))
    for g in range(_LG):
        sl = pl.ds(g * 16, 16)
        x = xv[sl]
        y = yv[sl]
        x0i = x.astype(jnp.int32)
        y0i = y.astype(jnp.int32)
        wx1 = x - x0i.astype(jnp.float32)
        wy1 = y - y0i.astype(jnp.float32)
        wx0 = 1.0 - wx1
        wy0 = 1.0 - wy1
        w00v[sl] = wy0 * wx0
        w01v[sl] = wy0 * wx1
        w10v[sl] = wy1 * wx0
        w11v[sl] = wy1 * wx1
        x0 = jnp.minimum(jnp.maximum(x0i, 0), _W - 1)
        x1 = jnp.minimum(jnp.maximum(x0i + 1, 0), _W - 1)
        y0 = jnp.minimum(jnp.maximum(y0i, 0), _H - 1)
        y1 = jnp.minimum(jnp.maximum(y0i + 1, 0), _H - 1)
        i00h, i01h, i10h, i11h = idx_half[g // (_LG // 2)]
        hsl = pl.ds((g % (_LG // 2)) * 16, 16)
        i00h[hsl] = y0 * _W + x0 + rbase
        i01h[hsl] = y0 * _W + x1 + rbase
        i10h[hsl] = y1 * _W + x0 + rbase
        i11h[hsl] = y1 * _W + x1 + rbase

    hp = _PPW // 2
    for h in range(2):
        i00h, i01h, i10h, i11h = idx_half[h]
        c0 = pltpu.async_copy(tab_hbm.at[i00h], r00, sem)
        c1 = pltpu.async_copy(tab_hbm.at[i01h], r01, sem)
        c2 = pltpu.async_copy(tab_hbm.at[i10h], r10, sem)
        c3 = pltpu.async_copy(tab_hbm.at[i11h], r11, sem)
        c0.wait()
        c1.wait()
        c2.wait()
        c3.wait()

        def blend_body(g2, _, h=h):
            wsl = pl.ds(h * hp + g2 * 16, 16)
            wv00 = w00v[wsl]
            wv01 = w01v[wsl]
            wv10 = w10v[wsl]
            wv11 = w11v[wsl]
            for k in range(16):
                j = g2 * 16 + k
                p = h * hp + j
                for cg in range(_C2D // 16):
                    cs = pl.ds(cg * 16, 16)
                    outv[p, cs] = (wv00[k] * r00[j, cs]
                                   + wv01[k] * r01[j, cs]
                                   + wv10[k] * r10[j, cs]
                                   + wv11[k] * r11[j, cs])
            return _

        lax.fori_loop(0, hp // 16, blend_body, 0)

    pltpu.sync_copy(outv, out_hbm.at[pl.ds(base, _PPW)])


def _fuse_kernel(f2pm_ref, f3ipm_ref, f2spm_ref, f3pm_ref,
                 mw_ref, mb_ref, a1w_ref, a1b_ref,
                 f2a1w_ref, f2a1b_ref, f2a2w_ref, f2a2b_ref,
                 f2midw_ref, f2evw_ref, f2odw_ref,
                 f3a2w_ref, f3a2b_ref, f3midw_ref, f3evw_ref, f3odw_ref,
                 out2d_ref, out3d_ref):
    f32 = jnp.float32
    a2 = _lrelu(jnp.dot(f2pm_ref[0], f2a1w_ref[:], preferred_element_type=f32)
                + f2a1b_ref[:])
    b2 = _lrelu(jnp.dot(f3ipm_ref[0], f2a2w_ref[:], preferred_element_type=f32)
                + f2a2b_ref[:])
    w = jnp.sum(a2 + b2, axis=0, keepdims=True) * (1.0 / _HW)
    r = jnp.maximum(jnp.dot(w, f2midw_ref[:], preferred_element_type=f32), 0.0)
    v0 = jax.nn.sigmoid(jnp.dot(r, f2evw_ref[:], preferred_element_type=f32))
    v1 = jax.nn.sigmoid(jnp.dot(r, f2odw_ref[:], preferred_element_type=f32))
    p0, p1 = _pair_softmax(v0, v1)
    out2d_ref[0] = jnp.transpose(a2 * p0 + b2 * p1)

    m = _lrelu(jnp.dot(f2spm_ref[0], mw_ref[:], preferred_element_type=f32)
               + mb_ref[:])
    a3 = _lrelu(jnp.dot(m, a1w_ref[:], preferred_element_type=f32)
                + a1b_ref[:])
    b3 = _lrelu(jnp.dot(f3pm_ref[0], f3a2w_ref[:], preferred_element_type=f32)
                + f3a2b_ref[:])
    w3 = jnp.sum(a3 + b3, axis=0, keepdims=True) * (1.0 / _N)
    r3 = jnp.maximum(jnp.dot(w3, f3midw_ref[:], preferred_element_type=f32),
                     0.0)
    v30 = jax.nn.sigmoid(jnp.dot(r3, f3evw_ref[:], preferred_element_type=f32))
    v31 = jax.nn.sigmoid(jnp.dot(r3, f3odw_ref[:], preferred_element_type=f32))
    p30, p31 = _pair_softmax(v30, v31)
    out3d_ref[0] = jnp.transpose(a3 * p30 + b3 * p31)


def _vspec(shape):
    nd = len(shape)
    return pl.BlockSpec(shape, lambda *_: (0,) * nd)


def kernel(uv, feat_2d, feat_3d, interp_out_W, interp_out_b, score1_W,
           score1_b, score2_W, score2_b, mlps3d_W, mlps3d_b, f2_a1_W, f2_a1_b,
           f2_a2_W, f2_a2_b, f2_mid_W, f2_out_W, f3_a1_W, f3_a1_b, f3_a2_W,
           f3_a2_b, f3_mid_W, f3_out_W):
    f32 = jnp.float32
    f2t = feat_2d.transpose(0, 2, 3, 1)
    f2pm = f2t.reshape(_BS, _HW, _C2D)
    f3pm = feat_3d.transpose(0, 2, 1)
    uvt = uv.transpose(0, 2, 1)
    cat = jnp.concatenate([uvt, f3pm], axis=2)
    s1 = jnp.concatenate([score1_W.reshape(1, 3),
                          score1_b.reshape(1, 1)], axis=1)
    s2w = score2_W.reshape(1, _C3D)
    s2b = score2_b.reshape(1, _C3D)

    f3i_pm = pl.pallas_call(
        _interp_kernel,
        grid=(_BS, _NC),
        in_specs=[
            pl.BlockSpec(memory_space=pltpu.SMEM),
            pl.BlockSpec((1, 2, _N), lambda b, c: (b, 0, 0)),
            pl.BlockSpec((1, _N, 2 + _C3D), lambda b, c: (b, 0, 0)),
            pl.BlockSpec((1, _C3D), lambda b, c: (0, 0)),
            pl.BlockSpec((1, _C3D), lambda b, c: (0, 0)),
            pl.BlockSpec((_C3D, _C3D), lambda b, c: (0, 0)),
            pl.BlockSpec((1, _C3D), lambda b, c: (0, 0)),
        ],
        out_specs=pl.BlockSpec((1, _P, _C3D), lambda b, c: (b, c, 0)),
        out_shape=jax.ShapeDtypeStruct((_BS, _HW, _C3D), f32),
    )(s1, uv, cat, s2w, s2b, interp_out_W.T,
      interp_out_b.reshape(1, _C3D))

    sc_bilinear = functools.partial(
        pl.kernel,
        mesh=plsc.VectorSubcoreMesh(core_axis_name="c", subcore_axis_name="s"),
        out_type=jax.ShapeDtypeStruct((_BS * _N, _C2D), f32),
        scratch_types=[
            pltpu.VMEM((_PPW,), f32), pltpu.VMEM((_PPW,), f32),
            pltpu.VMEM((_PPW,), f32), pltpu.VMEM((_PPW,), f32),
            pltpu.VMEM((_PPW,), f32), pltpu.VMEM((_PPW,), f32),
            pltpu.VMEM((_PPW // 2,), jnp.int32),
            pltpu.VMEM((_PPW // 2,), jnp.int32),
            pltpu.VMEM((_PPW // 2,), jnp.int32),
            pltpu.VMEM((_PPW // 2,), jnp.int32),
            pltpu.VMEM((_PPW // 2,), jnp.int32),
            pltpu.VMEM((_PPW // 2,), jnp.int32),
            pltpu.VMEM((_PPW // 2,), jnp.int32),
            pltpu.VMEM((_PPW // 2,), jnp.int32),
            pltpu.VMEM((_PPW // 2, 2 * _C2D), f32),
            pltpu.VMEM((_PPW // 2, 2 * _C2D), f32),
            pltpu.VMEM((_PPW // 2, 2 * _C2D), f32),
            pltpu.VMEM((_PPW // 2, 2 * _C2D), f32),
            pltpu.VMEM((_PPW, _C2D), f32),
            pltpu.SemaphoreType.DMA,
        ],
    )(_sc_bilinear_kernel)
    ux = uv[:, 0, :].reshape(_BS * _N)
    uy = uv[:, 1, :].reshape(_BS * _N)
    tab = jnp.concatenate(
        [f2t.reshape(_BS * _HW, _C2D),
         jnp.zeros((_BS * _HW, _C2D), f32)], axis=1)
    f2s_pm = sc_bilinear(ux, uy, tab).reshape(_BS, _N, _C2D)

    out2d_pm, out3d_pm = pl.pallas_call(
        _fuse_kernel,
        grid=(_BS,),
        in_specs=[
            pl.BlockSpec((1, _HW, _C2D), lambda b: (b, 0, 0)),
            pl.BlockSpec((1, _HW, _C3D), lambda b: (b, 0, 0)),
            pl.BlockSpec((1, _N, _C2D), lambda b: (b, 0, 0)),
            pl.BlockSpec((1, _N, _C3D), lambda b: (b, 0, 0)),
            _vspec((_C2D, _C2D)), _vspec((1, _C2D)),
            _vspec((_C2D, _C3D)), _vspec((1, _C3D)),
            _vspec((_C2D, _C2D)), _vspec((1, _C2D)),
            _vspec((_C3D, _C2D)), _vspec((1, _C2D)),
            _vspec((_C2D, _C2D // 2)),
            _vspec((_C2D // 2, _C2D)), _vspec((_C2D // 2, _C2D)),
            _vspec((_C3D, _C3D)), _vspec((1, _C3D)),
            _vspec((_C3D, _C3D // 2)),
            _vspec((_C3D // 2, _C3D)), _vspec((_C3D // 2, _C3D)),
        ],
        out_specs=[
            pl.BlockSpec((1, _C2D, _HW), lambda b: (b, 0, 0)),
            pl.BlockSpec((1, _C3D, _N), lambda b: (b, 0, 0)),
        ],
        out_shape=[
            jax.ShapeDtypeStruct((_BS, _C2D, _HW), f32),
            jax.ShapeDtypeStruct((_BS, _C3D, _N), f32),
        ],
    )(f2pm, f3i_pm, f2s_pm, f3pm,
      mlps3d_W.T, mlps3d_b.reshape(1, _C2D),
      f3_a1_W.T, f3_a1_b.reshape(1, _C3D),
      f2_a1_W.T, f2_a1_b.reshape(1, _C2D),
      f2_a2_W.T, f2_a2_b.reshape(1, _C2D),
      f2_mid_W.T, f2_out_W[0::2].T, f2_out_W[1::2].T,
      f3_a2_W.T, f3_a2_b.reshape(1, _C3D),
      f3_mid_W.T, f3_out_W[0::2].T, f3_out_W[1::2].T)

    return (out2d_pm.reshape(_BS, _C2D, _H, _W), out3d_pm)
